# baseline (speedup 1.0000x reference)
import dataclasses

import jax
import jax.numpy as jnp
from jax import lax
from jax.experimental import pallas as pl
from jax.experimental.pallas import tpu as pltpu
from jax.experimental.pallas import tpu_sc as plsc

N = 10000
E = 320000
HID = 128
NGRAPH = 64
TYPE_V = 64
TYPE_D = 16
NAME_D = 64

NSC = 2
NSUB = 16
LANES = 16
NW = NSC * NSUB

EC = 80
TCH = E // EC
E_PER_SC = E // NSC
NCHUNK_SC = E_PER_SC // EC
T_SUB = NCHUNK_SC // NSUB
NBUF = 4
RS = 624
TAIL = N - RS * NSUB

NAMC = 80
NAME_CHUNKS = N // NAMC

_mesh = plsc.VectorSubcoreMesh(core_axis_name="c", subcore_axis_name="s")

_sc_params = pltpu.CompilerParams()
_fields = pltpu.CompilerParams.__dataclass_fields__
if "needs_layout_passes" in _fields:
    _sc_params = dataclasses.replace(_sc_params, needs_layout_passes=False)
if "use_tc_tiling_on_sc" in _fields:
    _sc_params = dataclasses.replace(_sc_params, use_tc_tiling_on_sc=False)


def _sc_prep_body(ep_hbm, names_hbm, table_hbm, z16_hbm,
                  degp_hbm, nfeat_hbm,
                  idx_v, nrow_v, ep_v, deg_rows, deg_sh, sem,
                  e0, e1, e2, e3, s0, s1, s2, s3):
    cid = lax.axis_index("c")
    sid = lax.axis_index("s")
    wid = sid * NSC + cid
    esem = (e0, e1, e2, e3)
    ssem = (s0, s1, s2, s3)

    @pl.loop(wid, NAME_CHUNKS, step=NW)
    def _(j):
        base = j * NAMC
        pltpu.sync_copy(names_hbm.at[pl.ds(base, NAMC)], idx_v)
        pltpu.async_copy(table_hbm.at[idx_v], nrow_v, sem).wait()
        pltpu.sync_copy(nrow_v, nfeat_hbm.at[pl.ds(base, NAMC)])

    r0 = sid * RS
    pltpu.sync_copy(z16_hbm.at[pl.ds(r0, RS)], deg_sh.at[pl.ds(r0, RS)])

    @pl.when(sid == 0)
    def _():
        pltpu.sync_copy(z16_hbm.at[pl.ds(RS * NSUB, TAIL)],
                        deg_sh.at[pl.ds(RS * NSUB, TAIL)])

    plsc.subcore_barrier()

    j0 = cid * NCHUNK_SC + sid

    def fetch(t, b):
        pltpu.async_copy(ep_hbm.at[j0 + t * NSUB], ep_v.at[b], esem[b])

    def drain_scatter(b):
        pltpu.make_async_copy(deg_rows.at[b], deg_sh.at[ep_v.at[b, 1]],
                              ssem[b]).wait()

    def process(t, b, head):
        pltpu.make_async_copy(ep_hbm.at[j0], ep_v.at[b], esem[b]).wait()

        @pl.loop(0, EC, unroll=4)
        def _(e):
            deg_rows[b, e, :] = plsc.bitcast(
                plsc.load_gather(ep_v.at[b, 2],
                                 [jnp.full((LANES,), e, jnp.int32)]),
                jnp.float32)

        pltpu.async_copy(deg_rows.at[b], deg_sh.at[ep_v.at[b, 1]], ssem[b],
                         add=True)
        b2 = (b + 2) % NBUF
        if head:
            if t >= 2:
                drain_scatter(b2)
            fetch(t + 2, b2)
        else:
            @pl.when(t + 2 < T_SUB)
            def _():
                drain_scatter(b2)
                fetch(t + 2, b2)

    fetch(0, 0)
    fetch(1, 1)
    for t in range(NBUF):
        process(t, t, True)

    @pl.loop(1, T_SUB // NBUF)
    def _(q):
        t = q * NBUF
        for p in range(NBUF):
            process(t + p, p, False)

    for t in range((T_SUB // NBUF) * NBUF, T_SUB):
        process(t, t % NBUF, False)

    for b in range(NBUF):
        drain_scatter(b)

    plsc.subcore_barrier()
    pltpu.sync_copy(deg_sh.at[pl.ds(r0, RS)],
                    degp_hbm.at[cid, pl.ds(r0, RS)])

    @pl.when(sid == 0)
    def _():
        pltpu.sync_copy(deg_sh.at[pl.ds(RS * NSUB, TAIL)],
                        degp_hbm.at[cid, pl.ds(RS * NSUB, TAIL)])


_sc_prep = pl.kernel(
    _sc_prep_body,
    out_type=(jax.ShapeDtypeStruct((NSC, N, LANES), jnp.float32),
              jax.ShapeDtypeStruct((N, NAME_D), jnp.float32)),
    mesh=_mesh,
    scratch_types=[
        pltpu.VMEM((NAMC,), jnp.int32),
        pltpu.VMEM((NAMC, NAME_D), jnp.float32),
        pltpu.VMEM((NBUF, 3, EC), jnp.int32),
        pltpu.VMEM((NBUF, EC, LANES), jnp.float32),
        pltpu.VMEM_SHARED((N, LANES), jnp.float32),
        pltpu.SemaphoreType.DMA,
        pltpu.SemaphoreType.DMA,
        pltpu.SemaphoreType.DMA,
        pltpu.SemaphoreType.DMA,
        pltpu.SemaphoreType.DMA,
        pltpu.SemaphoreType.DMA,
        pltpu.SemaphoreType.DMA,
        pltpu.SemaphoreType.DMA,
        pltpu.SemaphoreType.DMA,
    ],
    compiler_params=_sc_params,
)


def _sc_agg_body(ep_hbm, h_hbm, z_hbm,
                 acc_hbm,
                 ep_v, rows_v, acc_sh,
                 g0, g1, g2, g3, s0, s1, s2, s3, e0, e1, e2, e3):
    cid = lax.axis_index("c")
    sid = lax.axis_index("s")
    gsem = (g0, g1, g2, g3)
    ssem = (s0, s1, s2, s3)
    esem = (e0, e1, e2, e3)

    r0 = sid * RS
    pltpu.sync_copy(z_hbm.at[pl.ds(r0, RS)], acc_sh.at[pl.ds(r0, RS)])

    @pl.when(sid == 0)
    def _():
        pltpu.sync_copy(z_hbm.at[pl.ds(RS * NSUB, TAIL)],
                        acc_sh.at[pl.ds(RS * NSUB, TAIL)])

    plsc.subcore_barrier()

    j0 = cid * NCHUNK_SC + sid

    def fetch_ep(t, b):
        pltpu.async_copy(ep_hbm.at[j0 + t * NSUB], ep_v.at[b], esem[b])

    def start_gather(b):
        pltpu.make_async_copy(ep_hbm.at[j0], ep_v.at[b], esem[b]).wait()
        pltpu.async_copy(h_hbm.at[ep_v.at[b, 0]], rows_v.at[b], gsem[b])

    def drain_scatter(b):
        pltpu.make_async_copy(rows_v.at[b], acc_sh.at[ep_v.at[b, 1]],
                              ssem[b]).wait()

    def process(t, b, head):
        pltpu.make_async_copy(h_hbm.at[ep_v.at[b, 0]], rows_v.at[b],
                              gsem[b]).wait()

        @pl.loop(0, EC, unroll=4)
        def _(e):
            w16 = plsc.bitcast(
                plsc.load_gather(ep_v.at[b, 2],
                                 [jnp.full((LANES,), e, jnp.int32)]),
                jnp.float32)
            for k in range(HID // LANES):
                rows_v[b, e, pl.ds(k * LANES, LANES)] = (
                    rows_v[b, e, pl.ds(k * LANES, LANES)] * w16)

        pltpu.async_copy(rows_v.at[b], acc_sh.at[ep_v.at[b, 1]], ssem[b],
                         add=True)
        b1 = (b + 1) % NBUF
        b2 = (b + 2) % NBUF
        if head:
            if t >= 2:
                drain_scatter(b2)
            fetch_ep(t + 2, b2)
            start_gather(b1)
        else:
            @pl.when(t + 2 < T_SUB)
            def _():
                drain_scatter(b2)
                fetch_ep(t + 2, b2)

            @pl.when(t + 1 < T_SUB)
            def _():
                start_gather(b1)

    fetch_ep(0, 0)
    fetch_ep(1, 1)
    start_gather(0)
    for t in range(NBUF):
        process(t, t, True)

    @pl.loop(1, T_SUB // NBUF)
    def _(q):
        t = q * NBUF
        for p in range(NBUF):
            process(t + p, p, False)

    for t in range((T_SUB // NBUF) * NBUF, T_SUB):
        process(t, t % NBUF, False)

    for b in range(NBUF):
        drain_scatter(b)

    plsc.subcore_barrier()
    pltpu.sync_copy(acc_sh.at[pl.ds(r0, RS)],
                    acc_hbm.at[cid, pl.ds(r0, RS)])

    @pl.when(sid == 0)
    def _():
        pltpu.sync_copy(acc_sh.at[pl.ds(RS * NSUB, TAIL)],
                        acc_hbm.at[cid, pl.ds(RS * NSUB, TAIL)])


_sc_agg = pl.kernel(
    _sc_agg_body,
    out_type=jax.ShapeDtypeStruct((NSC, N, HID), jnp.float32),
    mesh=_mesh,
    scratch_types=[
        pltpu.VMEM((NBUF, 3, EC), jnp.int32),
        pltpu.VMEM((NBUF, EC, HID), jnp.float32),
        pltpu.VMEM_SHARED((N, HID), jnp.float32),
    ] + [pltpu.SemaphoreType.DMA] * 12,
    compiler_params=_sc_params,
)


def _tc1_body(degp_ref, nf_ref, xt_ref, xb_ref, tt_ref, w1_ref,
              dinv_ref, h2_ref):
    deg = degp_ref[0, :, 0:1] + degp_ref[1, :, 0:1] + 1.0
    dinv = lax.rsqrt(deg)
    dinv_ref[...] = dinv
    w1 = w1_ref[...]
    type_proj = jnp.dot(tt_ref[...], w1[NAME_D:NAME_D + TYPE_D, :],
                        preferred_element_type=jnp.float32)
    oh = jnp.where(
        lax.broadcasted_iota(jnp.int32, (N, TYPE_V), 1) == xt_ref[...],
        1.0, 0.0)
    xw = (jnp.dot(nf_ref[...], w1[:NAME_D, :],
                  preferred_element_type=jnp.float32)
          + jnp.dot(oh, type_proj, preferred_element_type=jnp.float32)
          + jnp.dot(xb_ref[...], w1[NAME_D + TYPE_D:, :],
                    preferred_element_type=jnp.float32))
    h2_ref[...] = dinv * xw


def _tc2_body(acc_ref, h2_ref, dinv_ref, b_ref, w2_ref, out_ref):
    dinv = dinv_ref[...]
    a = jnp.maximum(
        dinv * (acc_ref[0] + acc_ref[1] + h2_ref[...]) + b_ref[...], 0.0)
    out_ref[...] = dinv * jnp.dot(a, w2_ref[...],
                                  preferred_element_type=jnp.float32)


def _tc3_body(acc_ref, h2_ref, dinv_ref, b_ref, batch_ref, wc_ref, bc_ref,
              out_ref):
    a = jnp.maximum(
        dinv_ref[...] * (acc_ref[0] + acc_ref[1] + h2_ref[...]) + b_ref[...],
        0.0)
    oh = jnp.where(
        lax.broadcasted_iota(jnp.int32, (NGRAPH, N), 0) == batch_ref[...],
        1.0, 0.0)
    sums = jnp.dot(oh, a, preferred_element_type=jnp.float32)
    cnts = jnp.sum(oh, axis=1, keepdims=True)
    pooled = sums / jnp.maximum(cnts, 1.0)
    out_ref[...] = (jnp.dot(pooled, wc_ref[...],
                            preferred_element_type=jnp.float32) + bc_ref[...])


def kernel(x_names, x_types, x_behaviors, edge_index, edge_weight, batch,
           name_table, type_table, W1, b1, W2, b2, Wc, bc):
    src = edge_index[0].astype(jnp.int32)
    dst = edge_index[1].astype(jnp.int32)
    names = x_names.astype(jnp.int32)
    ew_bits = lax.bitcast_convert_type(edge_weight.astype(jnp.float32),
                                       jnp.int32)
    epack = jnp.stack([src.reshape(TCH, EC), dst.reshape(TCH, EC),
                       ew_bits.reshape(TCH, EC)], axis=1)
    xt = x_types.astype(jnp.int32).reshape(N, 1)
    batch2 = batch.astype(jnp.int32).reshape(1, N)
    z16 = jnp.zeros((N, LANES), jnp.float32)
    z128 = jnp.zeros((N, HID), jnp.float32)
    ncls = Wc.shape[1]

    degp, nfeat = _sc_prep(epack, names, name_table.astype(jnp.float32),
                           z16)

    dinv, h2 = pl.pallas_call(
        _tc1_body,
        out_shape=(jax.ShapeDtypeStruct((N, 1), jnp.float32),
                   jax.ShapeDtypeStruct((N, HID), jnp.float32)),
    )(degp, nfeat, xt, x_behaviors.astype(jnp.float32),
      type_table.astype(jnp.float32), W1)

    acc1 = _sc_agg(epack, h2, z128)

    h2b = pl.pallas_call(
        _tc2_body,
        out_shape=jax.ShapeDtypeStruct((N, HID), jnp.float32),
    )(acc1, h2, dinv, b1.reshape(1, HID), W2)

    acc2 = _sc_agg(epack, h2b, z128)

    out = pl.pallas_call(
        _tc3_body,
        out_shape=jax.ShapeDtypeStruct((NGRAPH, ncls), jnp.float32),
    )(acc2, h2b, dinv, b2.reshape(1, HID), batch2, Wc, bc.reshape(1, ncls))
    return out


# speedup vs baseline: 1.4094x; 1.4094x over previous
import dataclasses

import jax
import jax.numpy as jnp
from jax import lax
from jax.experimental import pallas as pl
from jax.experimental.pallas import tpu as pltpu
from jax.experimental.pallas import tpu_sc as plsc

N = 10000
E = 320000
HID = 128
NGRAPH = 64
TYPE_V = 64
TYPE_D = 16
NAME_D = 64

NSC = 2
NSUB = 16
LANES = 16
NW = NSC * NSUB

EC = 80
TCH = E // EC
E_PER_SC = E // NSC
NCHUNK_SC = E_PER_SC // EC
T_SUB = NCHUNK_SC // NSUB
NBUF = 4
RS = 624
TAIL = N - RS * NSUB

NAMC = 80
NAME_CHUNKS = N // NAMC

_mesh = plsc.VectorSubcoreMesh(core_axis_name="c", subcore_axis_name="s")

_sc_params = pltpu.CompilerParams()
_fields = pltpu.CompilerParams.__dataclass_fields__
if "needs_layout_passes" in _fields:
    _sc_params = dataclasses.replace(_sc_params, needs_layout_passes=False)
if "use_tc_tiling_on_sc" in _fields:
    _sc_params = dataclasses.replace(_sc_params, use_tc_tiling_on_sc=False)


def _sc_prep_body(ep_hbm, names_hbm, table_hbm, z16_hbm,
                  degp_hbm, nfeat_hbm,
                  idx_v, nrow_v, ep_v, deg_rows, deg_sh, sem,
                  e0, e1, e2, e3, s0, s1, s2, s3):
    cid = lax.axis_index("c")
    sid = lax.axis_index("s")
    wid = sid * NSC + cid
    esem = (e0, e1, e2, e3)
    ssem = (s0, s1, s2, s3)

    @pl.loop(wid, NAME_CHUNKS, step=NW)
    def _(j):
        base = j * NAMC
        pltpu.sync_copy(names_hbm.at[pl.ds(base, NAMC)], idx_v)
        pltpu.async_copy(table_hbm.at[idx_v], nrow_v, sem).wait()
        pltpu.sync_copy(nrow_v, nfeat_hbm.at[pl.ds(base, NAMC)])

    r0 = sid * RS
    pltpu.sync_copy(z16_hbm.at[pl.ds(r0, RS)], deg_sh.at[pl.ds(r0, RS)])

    @pl.when(sid == 0)
    def _():
        pltpu.sync_copy(z16_hbm.at[pl.ds(RS * NSUB, TAIL)],
                        deg_sh.at[pl.ds(RS * NSUB, TAIL)])

    plsc.subcore_barrier()

    j0 = cid * NCHUNK_SC + sid

    def fetch(t, b):
        pltpu.async_copy(ep_hbm.at[j0 + t * NSUB], ep_v.at[b], esem[b])

    def drain_scatter(b):
        pltpu.make_async_copy(deg_rows.at[b], deg_sh.at[ep_v.at[b, 1]],
                              ssem[b]).wait()

    def process(t, b, head):
        pltpu.make_async_copy(ep_hbm.at[j0], ep_v.at[b], esem[b]).wait()

        @pl.loop(0, EC, unroll=4)
        def _(e):
            deg_rows[b, e, :] = plsc.bitcast(
                plsc.load_gather(ep_v.at[b, 2],
                                 [jnp.full((LANES,), e, jnp.int32)]),
                jnp.float32)

        pltpu.async_copy(deg_rows.at[b], deg_sh.at[ep_v.at[b, 1]], ssem[b],
                         add=True)
        b2 = (b + 2) % NBUF
        if head:
            if t >= 2:
                drain_scatter(b2)
            fetch(t + 2, b2)
        else:
            @pl.when(t + 2 < T_SUB)
            def _():
                drain_scatter(b2)
                fetch(t + 2, b2)

    fetch(0, 0)
    fetch(1, 1)
    for t in range(NBUF):
        process(t, t, True)

    @pl.loop(1, T_SUB // NBUF)
    def _(q):
        t = q * NBUF
        for p in range(NBUF):
            process(t + p, p, False)

    for t in range((T_SUB // NBUF) * NBUF, T_SUB):
        process(t, t % NBUF, False)

    for b in range(NBUF):
        drain_scatter(b)

    plsc.subcore_barrier()
    pltpu.sync_copy(deg_sh.at[pl.ds(r0, RS)],
                    degp_hbm.at[cid, pl.ds(r0, RS)])

    @pl.when(sid == 0)
    def _():
        pltpu.sync_copy(deg_sh.at[pl.ds(RS * NSUB, TAIL)],
                        degp_hbm.at[cid, pl.ds(RS * NSUB, TAIL)])


_sc_prep = pl.kernel(
    _sc_prep_body,
    out_type=(jax.ShapeDtypeStruct((NSC, N, LANES), jnp.float32),
              jax.ShapeDtypeStruct((N, NAME_D), jnp.float32)),
    mesh=_mesh,
    scratch_types=[
        pltpu.VMEM((NAMC,), jnp.int32),
        pltpu.VMEM((NAMC, NAME_D), jnp.float32),
        pltpu.VMEM((NBUF, 3, EC), jnp.int32),
        pltpu.VMEM((NBUF, EC, LANES), jnp.float32),
        pltpu.VMEM_SHARED((N, LANES), jnp.float32),
        pltpu.SemaphoreType.DMA,
        pltpu.SemaphoreType.DMA,
        pltpu.SemaphoreType.DMA,
        pltpu.SemaphoreType.DMA,
        pltpu.SemaphoreType.DMA,
        pltpu.SemaphoreType.DMA,
        pltpu.SemaphoreType.DMA,
        pltpu.SemaphoreType.DMA,
        pltpu.SemaphoreType.DMA,
    ],
    compiler_params=_sc_params,
)


NEP = 8


def _sc_agg_body(ep_hbm, h_hbm, z_hbm,
                 acc_hbm,
                 ep_v, rows_v, acc_sh,
                 g0, g1, g2, g3, s0, s1, s2, s3,
                 e0, e1, e2, e3, e4, e5, e6, e7):
    cid = lax.axis_index("c")
    sid = lax.axis_index("s")
    gsem = (g0, g1, g2, g3)
    ssem = (s0, s1, s2, s3)
    esem = (e0, e1, e2, e3, e4, e5, e6, e7)

    r0 = sid * RS
    pltpu.sync_copy(z_hbm.at[pl.ds(r0, RS)], acc_sh.at[pl.ds(r0, RS)])

    @pl.when(sid == 0)
    def _():
        pltpu.sync_copy(z_hbm.at[pl.ds(RS * NSUB, TAIL)],
                        acc_sh.at[pl.ds(RS * NSUB, TAIL)])

    plsc.subcore_barrier()

    j0 = cid * NCHUNK_SC + sid

    def fetch_ep(t, c):
        pltpu.async_copy(ep_hbm.at[j0 + t * NSUB], ep_v.at[c], esem[c])

    def start_gather(b, c):
        pltpu.make_async_copy(ep_hbm.at[j0], ep_v.at[c], esem[c]).wait()
        pltpu.async_copy(h_hbm.at[ep_v.at[c, 0]], rows_v.at[b], gsem[b])

    def drain_scatter(b, c):
        pltpu.make_async_copy(rows_v.at[b], acc_sh.at[ep_v.at[c, 1]],
                              ssem[b]).wait()

    def process(t, b, c, head):
        pltpu.make_async_copy(h_hbm.at[ep_v.at[c, 0]], rows_v.at[b],
                              gsem[b]).wait()

        @pl.loop(0, EC, unroll=4)
        def _(e):
            w16 = plsc.bitcast(
                plsc.load_gather(ep_v.at[c, 2],
                                 [jnp.full((LANES,), e, jnp.int32)]),
                jnp.float32)
            for k in range(HID // LANES):
                rows_v[b, e, pl.ds(k * LANES, LANES)] = (
                    rows_v[b, e, pl.ds(k * LANES, LANES)] * w16)

        pltpu.async_copy(rows_v.at[b], acc_sh.at[ep_v.at[c, 1]], ssem[b],
                         add=True)
        b2 = (b + 2) % NBUF
        c2 = (c + 2) % NEP
        c4 = (c + 4) % NEP
        cm2 = (c - 2) % NEP
        if head:
            if t >= 2:
                drain_scatter(b2, cm2)
            if t + 4 < T_SUB:
                fetch_ep(t + 4, c4)
            if t + 2 < T_SUB:
                start_gather(b2, c2)
        else:
            @pl.when(t + 4 < T_SUB)
            def _():
                fetch_ep(t + 4, c4)

            @pl.when(t + 2 < T_SUB)
            def _():
                drain_scatter(b2, cm2)
                start_gather(b2, c2)

    for t in range(4):
        fetch_ep(t, t)
    start_gather(0, 0)
    start_gather(1, 1)
    for t in range(NEP):
        process(t, t % NBUF, t, True)

    @pl.loop(1, T_SUB // NEP)
    def _(q):
        t0 = q * NEP
        for p in range(NEP):
            process(t0 + p, p % NBUF, p, False)

    for t in range((T_SUB // NEP) * NEP, T_SUB):
        process(t, t % NBUF, t % NEP, False)

    drain_scatter(1, (T_SUB - 4) % NEP)
    drain_scatter(2, (T_SUB - 3) % NEP)
    drain_scatter(3, (T_SUB - 2) % NEP)
    drain_scatter(0, (T_SUB - 1) % NEP)

    plsc.subcore_barrier()
    pltpu.sync_copy(acc_sh.at[pl.ds(r0, RS)],
                    acc_hbm.at[cid, pl.ds(r0, RS)])

    @pl.when(sid == 0)
    def _():
        pltpu.sync_copy(acc_sh.at[pl.ds(RS * NSUB, TAIL)],
                        acc_hbm.at[cid, pl.ds(RS * NSUB, TAIL)])


_sc_agg = pl.kernel(
    _sc_agg_body,
    out_type=jax.ShapeDtypeStruct((NSC, N, HID), jnp.float32),
    mesh=_mesh,
    scratch_types=[
        pltpu.VMEM((NEP, 3, EC), jnp.int32),
        pltpu.VMEM((NBUF, EC, HID), jnp.float32),
        pltpu.VMEM_SHARED((N, HID), jnp.float32),
    ] + [pltpu.SemaphoreType.DMA] * 16,
    compiler_params=_sc_params,
)


def _tc1_body(degp_ref, nf_ref, xt_ref, xb_ref, tt_ref, w1_ref,
              dinv_ref, h2_ref):
    deg = degp_ref[0, :, 0:1] + degp_ref[1, :, 0:1] + 1.0
    dinv = lax.rsqrt(deg)
    dinv_ref[...] = dinv
    w1 = w1_ref[...]
    type_proj = jnp.dot(tt_ref[...], w1[NAME_D:NAME_D + TYPE_D, :],
                        preferred_element_type=jnp.float32)
    oh = jnp.where(
        lax.broadcasted_iota(jnp.int32, (N, TYPE_V), 1) == xt_ref[...],
        1.0, 0.0)
    xw = (jnp.dot(nf_ref[...], w1[:NAME_D, :],
                  preferred_element_type=jnp.float32)
          + jnp.dot(oh, type_proj, preferred_element_type=jnp.float32)
          + jnp.dot(xb_ref[...], w1[NAME_D + TYPE_D:, :],
                    preferred_element_type=jnp.float32))
    h2_ref[...] = dinv * xw


def _tc2_body(acc_ref, h2_ref, dinv_ref, b_ref, w2_ref, out_ref):
    dinv = dinv_ref[...]
    a = jnp.maximum(
        dinv * (acc_ref[0] + acc_ref[1] + h2_ref[...]) + b_ref[...], 0.0)
    out_ref[...] = dinv * jnp.dot(a, w2_ref[...],
                                  preferred_element_type=jnp.float32)


def _tc3_body(acc_ref, h2_ref, dinv_ref, b_ref, batch_ref, wc_ref, bc_ref,
              out_ref):
    a = jnp.maximum(
        dinv_ref[...] * (acc_ref[0] + acc_ref[1] + h2_ref[...]) + b_ref[...],
        0.0)
    oh = jnp.where(
        lax.broadcasted_iota(jnp.int32, (NGRAPH, N), 0) == batch_ref[...],
        1.0, 0.0)
    sums = jnp.dot(oh, a, preferred_element_type=jnp.float32)
    cnts = jnp.sum(oh, axis=1, keepdims=True)
    pooled = sums / jnp.maximum(cnts, 1.0)
    out_ref[...] = (jnp.dot(pooled, wc_ref[...],
                            preferred_element_type=jnp.float32) + bc_ref[...])


def kernel(x_names, x_types, x_behaviors, edge_index, edge_weight, batch,
           name_table, type_table, W1, b1, W2, b2, Wc, bc):
    src = edge_index[0].astype(jnp.int32)
    dst = edge_index[1].astype(jnp.int32)
    names = x_names.astype(jnp.int32)
    ew_bits = lax.bitcast_convert_type(edge_weight.astype(jnp.float32),
                                       jnp.int32)
    epack = jnp.stack([src.reshape(TCH, EC), dst.reshape(TCH, EC),
                       ew_bits.reshape(TCH, EC)], axis=1)
    xt = x_types.astype(jnp.int32).reshape(N, 1)
    batch2 = batch.astype(jnp.int32).reshape(1, N)
    z16 = jnp.zeros((N, LANES), jnp.float32)
    z128 = jnp.zeros((N, HID), jnp.float32)
    ncls = Wc.shape[1]

    degp, nfeat = _sc_prep(epack, names, name_table.astype(jnp.float32),
                           z16)

    dinv, h2 = pl.pallas_call(
        _tc1_body,
        out_shape=(jax.ShapeDtypeStruct((N, 1), jnp.float32),
                   jax.ShapeDtypeStruct((N, HID), jnp.float32)),
    )(degp, nfeat, xt, x_behaviors.astype(jnp.float32),
      type_table.astype(jnp.float32), W1)

    acc1 = _sc_agg(epack, h2, z128)

    h2b = pl.pallas_call(
        _tc2_body,
        out_shape=jax.ShapeDtypeStruct((N, HID), jnp.float32),
    )(acc1, h2, dinv, b1.reshape(1, HID), W2)

    acc2 = _sc_agg(epack, h2b, z128)

    out = pl.pallas_call(
        _tc3_body,
        out_shape=jax.ShapeDtypeStruct((NGRAPH, ncls), jnp.float32),
    )(acc2, h2b, dinv, b2.reshape(1, HID), batch2, Wc, bc.reshape(1, ncls))
    return out


# speedup vs baseline: 1.4719x; 1.0444x over previous
import dataclasses

import jax
import jax.numpy as jnp
from jax import lax
from jax.experimental import pallas as pl
from jax.experimental.pallas import tpu as pltpu
from jax.experimental.pallas import tpu_sc as plsc

N = 10000
E = 320000
HID = 128
NGRAPH = 64
TYPE_V = 64
TYPE_D = 16
NAME_D = 64

NSC = 2
NSUB = 16
LANES = 16
NW = NSC * NSUB

EC = 80
E_PER_SC = E // NSC
NCHUNK_SC = E_PER_SC // EC
T_SUB = NCHUNK_SC // NSUB
NBUF = 4
NEP = 8
TAIL_START = (T_SUB // NEP) * NEP
N_GROUPS = T_SUB // NEP

RS = 624
TAIL = N - RS * NSUB

NAMC = 80
NAME_CHUNKS = N // NAMC

_mesh = plsc.VectorSubcoreMesh(core_axis_name="c", subcore_axis_name="s")

_sc_params = pltpu.CompilerParams()
_fields = pltpu.CompilerParams.__dataclass_fields__
if "needs_layout_passes" in _fields:
    _sc_params = dataclasses.replace(_sc_params, needs_layout_passes=False)
if "use_tc_tiling_on_sc" in _fields:
    _sc_params = dataclasses.replace(_sc_params, use_tc_tiling_on_sc=False)


def _splat(e):
    return jnp.full((LANES,), e, jnp.int32)


def _sc_prep_body(ei_hbm, ew_hbm, names_hbm, table_hbm, z16_hbm,
                  degp_hbm, nfeat_hbm,
                  idx_v, nrow_v, dst_v, ew_v, deg_rows, deg_sh, sem,
                  e0, e1, e2, e3, s0, s1, s2, s3):
    cid = lax.axis_index("c")
    sid = lax.axis_index("s")
    wid = sid * NSC + cid
    esem = (e0, e1, e2, e3)
    ssem = (s0, s1, s2, s3)

    @pl.loop(wid, NAME_CHUNKS, step=NW)
    def _(j):
        base = j * NAMC
        pltpu.sync_copy(names_hbm.at[pl.ds(base, NAMC)], idx_v)
        pltpu.async_copy(table_hbm.at[idx_v], nrow_v, sem).wait()
        pltpu.sync_copy(nrow_v, nfeat_hbm.at[pl.ds(base, NAMC)])

    r0 = sid * RS
    pltpu.sync_copy(z16_hbm.at[pl.ds(r0, RS)], deg_sh.at[pl.ds(r0, RS)])

    @pl.when(sid == 0)
    def _():
        pltpu.sync_copy(z16_hbm.at[pl.ds(RS * NSUB, TAIL)],
                        deg_sh.at[pl.ds(RS * NSUB, TAIL)])

    plsc.subcore_barrier()

    j0 = cid * NCHUNK_SC + sid

    def fetch(t, b):
        base = (j0 + t * NSUB) * EC
        pltpu.async_copy(ei_hbm.at[1, pl.ds(base, EC)], dst_v.at[b], esem[b])
        pltpu.async_copy(ew_hbm.at[pl.ds(base, EC)], ew_v.at[b], esem[b])

    def wait_fetch(b):
        pltpu.make_async_copy(ei_hbm.at[1, pl.ds(0, EC)], dst_v.at[b],
                              esem[b]).wait()
        pltpu.make_async_copy(ew_hbm.at[pl.ds(0, EC)], ew_v.at[b],
                              esem[b]).wait()

    def drain_scatter(b):
        pltpu.make_async_copy(deg_rows.at[b], deg_sh.at[dst_v.at[b]],
                              ssem[b]).wait()

    def process(t, b, drain, fetch_next):
        wait_fetch(b)

        @pl.loop(0, EC, unroll=4)
        def _(e):
            deg_rows[b, e, :] = plsc.load_gather(ew_v.at[b], [_splat(e)])

        pltpu.async_copy(deg_rows.at[b], deg_sh.at[dst_v.at[b]], ssem[b],
                         add=True)
        b2 = (b + 2) % NBUF
        if drain:
            drain_scatter(b2)
        if fetch_next:
            fetch(t + 2, b2)

    fetch(0, 0)
    fetch(1, 1)
    process(0, 0, False, True)
    process(1, 1, False, True)
    process(2, 2, True, True)
    process(3, 3, True, True)

    @pl.loop(1, (TAIL_START // NBUF))
    def _(q):
        t = q * NBUF
        for p in range(NBUF):
            process(t + p, p, True, True)

    for t in range(TAIL_START, T_SUB):
        process(t, t % NBUF, t + 2 < T_SUB, t + 2 < T_SUB)

    for b in range(NBUF):
        drain_scatter(b)

    plsc.subcore_barrier()
    pltpu.sync_copy(deg_sh.at[pl.ds(r0, RS)],
                    degp_hbm.at[cid, pl.ds(r0, RS)])

    @pl.when(sid == 0)
    def _():
        pltpu.sync_copy(deg_sh.at[pl.ds(RS * NSUB, TAIL)],
                        degp_hbm.at[cid, pl.ds(RS * NSUB, TAIL)])


_sc_prep = pl.kernel(
    _sc_prep_body,
    out_type=(jax.ShapeDtypeStruct((NSC, N, LANES), jnp.float32),
              jax.ShapeDtypeStruct((N, NAME_D), jnp.float32)),
    mesh=_mesh,
    scratch_types=[
        pltpu.VMEM((NAMC,), jnp.int32),
        pltpu.VMEM((NAMC, NAME_D), jnp.float32),
        pltpu.VMEM((NBUF, EC), jnp.int32),
        pltpu.VMEM((NBUF, EC), jnp.float32),
        pltpu.VMEM((NBUF, EC, LANES), jnp.float32),
        pltpu.VMEM_SHARED((N, LANES), jnp.float32),
    ] + [pltpu.SemaphoreType.DMA] * 9,
    compiler_params=_sc_params,
)


def _sc_agg_body(ei_hbm, ew_hbm, h_hbm, z_hbm,
                 acc_hbm,
                 ep_v, ew_v, rows_v, acc_sh,
                 g0, g1, g2, g3, s0, s1, s2, s3,
                 e0, e1, e2, e3, e4, e5, e6, e7):
    cid = lax.axis_index("c")
    sid = lax.axis_index("s")
    gsem = (g0, g1, g2, g3)
    ssem = (s0, s1, s2, s3)
    esem = (e0, e1, e2, e3, e4, e5, e6, e7)

    r0 = sid * RS
    pltpu.sync_copy(z_hbm.at[pl.ds(r0, RS)], acc_sh.at[pl.ds(r0, RS)])

    @pl.when(sid == 0)
    def _():
        pltpu.sync_copy(z_hbm.at[pl.ds(RS * NSUB, TAIL)],
                        acc_sh.at[pl.ds(RS * NSUB, TAIL)])

    plsc.subcore_barrier()

    j0 = cid * NCHUNK_SC + sid

    def fetch_ep(t, c):
        base = (j0 + t * NSUB) * EC
        pltpu.async_copy(ei_hbm.at[0, pl.ds(base, EC)], ep_v.at[c, 0],
                         esem[c])
        pltpu.async_copy(ei_hbm.at[1, pl.ds(base, EC)], ep_v.at[c, 1],
                         esem[c])
        pltpu.async_copy(ew_hbm.at[pl.ds(base, EC)], ew_v.at[c], esem[c])

    def start_gather(b, c):
        pltpu.make_async_copy(ei_hbm.at[0, pl.ds(0, EC)], ep_v.at[c, 0],
                              esem[c]).wait()
        pltpu.make_async_copy(ei_hbm.at[1, pl.ds(0, EC)], ep_v.at[c, 1],
                              esem[c]).wait()
        pltpu.make_async_copy(ew_hbm.at[pl.ds(0, EC)], ew_v.at[c],
                              esem[c]).wait()
        pltpu.async_copy(h_hbm.at[ep_v.at[c, 0]], rows_v.at[b], gsem[b])

    def drain_scatter(b, c):
        pltpu.make_async_copy(rows_v.at[b], acc_sh.at[ep_v.at[c, 1]],
                              ssem[b]).wait()

    def process(t, b, c, drain, fetch_next, gather_next):
        pltpu.make_async_copy(h_hbm.at[ep_v.at[c, 0]], rows_v.at[b],
                              gsem[b]).wait()

        @pl.loop(0, EC, unroll=4)
        def _(e):
            w16 = plsc.load_gather(ew_v.at[c], [_splat(e)])
            for k in range(HID // LANES):
                rows_v[b, e, pl.ds(k * LANES, LANES)] = (
                    rows_v[b, e, pl.ds(k * LANES, LANES)] * w16)

        pltpu.async_copy(rows_v.at[b], acc_sh.at[ep_v.at[c, 1]], ssem[b],
                         add=True)
        b2 = (b + 2) % NBUF
        if fetch_next:
            fetch_ep(t + 4, (c + 4) % NEP)
        if drain:
            drain_scatter(b2, (c - 2) % NEP)
        if gather_next:
            start_gather(b2, (c + 2) % NEP)

    for t in range(4):
        fetch_ep(t, t)
    start_gather(0, 0)
    start_gather(1, 1)
    for t in range(NEP):
        process(t, t % NBUF, t, t >= 2, True, True)

    @pl.loop(1, N_GROUPS)
    def _(q):
        t0 = q * NEP
        for p in range(NEP):
            process(t0 + p, p % NBUF, p, True, True, True)

    for t in range(TAIL_START, T_SUB):
        process(t, t % NBUF, t % NEP,
                t + 2 < T_SUB, t + 4 < T_SUB, t + 2 < T_SUB)

    for t in range(T_SUB - 4, T_SUB):
        drain_scatter(t % NBUF, t % NEP)

    plsc.subcore_barrier()
    pltpu.sync_copy(acc_sh.at[pl.ds(r0, RS)],
                    acc_hbm.at[cid, pl.ds(r0, RS)])

    @pl.when(sid == 0)
    def _():
        pltpu.sync_copy(acc_sh.at[pl.ds(RS * NSUB, TAIL)],
                        acc_hbm.at[cid, pl.ds(RS * NSUB, TAIL)])


_sc_agg = pl.kernel(
    _sc_agg_body,
    out_type=jax.ShapeDtypeStruct((NSC, N, HID), jnp.float32),
    mesh=_mesh,
    scratch_types=[
        pltpu.VMEM((NEP, 2, EC), jnp.int32),
        pltpu.VMEM((NEP, EC), jnp.float32),
        pltpu.VMEM((NBUF, EC, HID), jnp.float32),
        pltpu.VMEM_SHARED((N, HID), jnp.float32),
    ] + [pltpu.SemaphoreType.DMA] * 16,
    compiler_params=_sc_params,
)


def _tc1_body(degp_ref, nf_ref, xt_ref, xb_ref, tt_ref, w1_ref,
              dinv_ref, h2_ref):
    deg = degp_ref[0, :, 0:1] + degp_ref[1, :, 0:1] + 1.0
    dinv = lax.rsqrt(deg)
    dinv_ref[...] = dinv
    w1 = w1_ref[...]
    type_proj = jnp.dot(tt_ref[...], w1[NAME_D:NAME_D + TYPE_D, :],
                        preferred_element_type=jnp.float32)
    oh = jnp.where(
        lax.broadcasted_iota(jnp.int32, (N, TYPE_V), 1) == xt_ref[...],
        1.0, 0.0)
    xw = (jnp.dot(nf_ref[...], w1[:NAME_D, :],
                  preferred_element_type=jnp.float32)
          + jnp.dot(oh, type_proj, preferred_element_type=jnp.float32)
          + jnp.dot(xb_ref[...], w1[NAME_D + TYPE_D:, :],
                    preferred_element_type=jnp.float32))
    h2_ref[...] = dinv * xw


def _tc2_body(acc_ref, h2_ref, dinv_ref, b_ref, w2_ref, out_ref):
    dinv = dinv_ref[...]
    a = jnp.maximum(
        dinv * (acc_ref[0] + acc_ref[1] + h2_ref[...]) + b_ref[...], 0.0)
    out_ref[...] = dinv * jnp.dot(a, w2_ref[...],
                                  preferred_element_type=jnp.float32)


def _tc3_body(acc_ref, h2_ref, dinv_ref, b_ref, batch_ref, wc_ref, bc_ref,
              out_ref):
    a = jnp.maximum(
        dinv_ref[...] * (acc_ref[0] + acc_ref[1] + h2_ref[...]) + b_ref[...],
        0.0)
    oh = jnp.where(
        lax.broadcasted_iota(jnp.int32, (NGRAPH, N), 0) == batch_ref[...],
        1.0, 0.0)
    sums = jnp.dot(oh, a, preferred_element_type=jnp.float32)
    cnts = jnp.sum(oh, axis=1, keepdims=True)
    pooled = sums / jnp.maximum(cnts, 1.0)
    out_ref[...] = (jnp.dot(pooled, wc_ref[...],
                            preferred_element_type=jnp.float32) + bc_ref[...])


def kernel(x_names, x_types, x_behaviors, edge_index, edge_weight, batch,
           name_table, type_table, W1, b1, W2, b2, Wc, bc):
    ei = edge_index.astype(jnp.int32)
    ew = edge_weight.astype(jnp.float32)
    names = x_names.astype(jnp.int32)
    xt = x_types.astype(jnp.int32).reshape(N, 1)
    batch2 = batch.astype(jnp.int32).reshape(1, N)
    z16 = jnp.zeros((N, LANES), jnp.float32)
    z128 = jnp.zeros((N, HID), jnp.float32)
    ncls = Wc.shape[1]

    degp, nfeat = _sc_prep(ei, ew, names, name_table.astype(jnp.float32),
                           z16)

    dinv, h2 = pl.pallas_call(
        _tc1_body,
        out_shape=(jax.ShapeDtypeStruct((N, 1), jnp.float32),
                   jax.ShapeDtypeStruct((N, HID), jnp.float32)),
    )(degp, nfeat, xt, x_behaviors.astype(jnp.float32),
      type_table.astype(jnp.float32), W1)

    acc1 = _sc_agg(ei, ew, h2, z128)

    h2b = pl.pallas_call(
        _tc2_body,
        out_shape=jax.ShapeDtypeStruct((N, HID), jnp.float32),
    )(acc1, h2, dinv, b1.reshape(1, HID), W2)

    acc2 = _sc_agg(ei, ew, h2b, z128)

    out = pl.pallas_call(
        _tc3_body,
        out_shape=jax.ShapeDtypeStruct((NGRAPH, ncls), jnp.float32),
    )(acc2, h2b, dinv, b2.reshape(1, HID), batch2, Wc, bc.reshape(1, ncls))
    return out


# speedup vs baseline: 1.6408x; 1.1147x over previous
import dataclasses

import jax
import jax.numpy as jnp
from jax import lax
from jax.experimental import pallas as pl
from jax.experimental.pallas import tpu as pltpu
from jax.experimental.pallas import tpu_sc as plsc

N = 10000
E = 320000
HID = 128
NGRAPH = 64
TYPE_V = 64
TYPE_D = 16
NAME_D = 64

NSC = 2
NSUB = 16
LANES = 16
NW = NSC * NSUB

EC = 80
E_PER_SC = E // NSC
NCHUNK_SC = E_PER_SC // EC
T_SUB = NCHUNK_SC // NSUB
NBUF = 4
NEP = 8
TAIL_START = (T_SUB // NEP) * NEP
N_GROUPS = T_SUB // NEP

RS = 624
TAIL = N - RS * NSUB

NAMC = 80
NAME_CHUNKS = N // NAMC

_mesh = plsc.VectorSubcoreMesh(core_axis_name="c", subcore_axis_name="s")

_sc_params = pltpu.CompilerParams()
_fields = pltpu.CompilerParams.__dataclass_fields__
if "needs_layout_passes" in _fields:
    _sc_params = dataclasses.replace(_sc_params, needs_layout_passes=False)
if "use_tc_tiling_on_sc" in _fields:
    _sc_params = dataclasses.replace(_sc_params, use_tc_tiling_on_sc=False)


def _splat(e):
    return jnp.full((LANES,), e, jnp.int32)


_GDN = lax.GatherDimensionNumbers(
    offset_dims=(), collapsed_slice_dims=(0,), start_index_map=(0,))


def _bcast_lane(vec, j):
    return lax.gather(vec, jnp.full((LANES, 1), j, jnp.int32), _GDN,
                      slice_sizes=(1,),
                      mode=lax.GatherScatterMode.PROMISE_IN_BOUNDS)


def _sc_prep_body(ei_hbm, ew_hbm, names_hbm, table_hbm, z16_hbm,
                  degp_hbm, nfeat_hbm,
                  idx_v, nrow_v, dst_v, ew_v, deg_rows, deg_sh, sem,
                  e0, e1, e2, e3, s0, s1, s2, s3):
    cid = lax.axis_index("c")
    sid = lax.axis_index("s")
    wid = sid * NSC + cid
    esem = (e0, e1, e2, e3)
    ssem = (s0, s1, s2, s3)

    @pl.loop(wid, NAME_CHUNKS, step=NW)
    def _(j):
        base = j * NAMC
        pltpu.sync_copy(names_hbm.at[pl.ds(base, NAMC)], idx_v)
        pltpu.async_copy(table_hbm.at[idx_v], nrow_v, sem).wait()
        pltpu.sync_copy(nrow_v, nfeat_hbm.at[pl.ds(base, NAMC)])

    r0 = sid * RS
    pltpu.sync_copy(z16_hbm.at[pl.ds(r0, RS)], deg_sh.at[pl.ds(r0, RS)])

    @pl.when(sid == 0)
    def _():
        pltpu.sync_copy(z16_hbm.at[pl.ds(RS * NSUB, TAIL)],
                        deg_sh.at[pl.ds(RS * NSUB, TAIL)])

    plsc.subcore_barrier()

    j0 = cid * NCHUNK_SC + sid

    def fetch(t, b):
        base = (j0 + t * NSUB) * EC
        pltpu.async_copy(ei_hbm.at[1, pl.ds(base, EC)], dst_v.at[b], esem[b])
        pltpu.async_copy(ew_hbm.at[pl.ds(base, EC)], ew_v.at[b], esem[b])

    def wait_fetch(b):
        pltpu.make_async_copy(ei_hbm.at[1, pl.ds(0, EC)], dst_v.at[b],
                              esem[b]).wait()
        pltpu.make_async_copy(ew_hbm.at[pl.ds(0, EC)], ew_v.at[b],
                              esem[b]).wait()

    def drain_scatter(b):
        pltpu.make_async_copy(deg_rows.at[b], deg_sh.at[dst_v.at[b]],
                              ssem[b]).wait()

    def process(t, b, drain, fetch_next):
        wait_fetch(b)

        @pl.loop(0, EC // LANES)
        def _(g):
            w_grp = ew_v[b, pl.ds(g * LANES, LANES)]
            for j in range(LANES):
                deg_rows[b, g * LANES + j, :] = _bcast_lane(w_grp, j)

        pltpu.async_copy(deg_rows.at[b], deg_sh.at[dst_v.at[b]], ssem[b],
                         add=True)
        b2 = (b + 2) % NBUF
        if drain:
            drain_scatter(b2)
        if fetch_next:
            fetch(t + 2, b2)

    fetch(0, 0)
    fetch(1, 1)
    process(0, 0, False, True)
    process(1, 1, False, True)
    process(2, 2, True, True)
    process(3, 3, True, True)

    @pl.loop(1, (TAIL_START // NBUF))
    def _(q):
        t = q * NBUF
        for p in range(NBUF):
            process(t + p, p, True, True)

    for t in range(TAIL_START, T_SUB):
        process(t, t % NBUF, t + 2 < T_SUB, t + 2 < T_SUB)

    for b in range(NBUF):
        drain_scatter(b)

    plsc.subcore_barrier()
    pltpu.sync_copy(deg_sh.at[pl.ds(r0, RS)],
                    degp_hbm.at[cid, pl.ds(r0, RS)])

    @pl.when(sid == 0)
    def _():
        pltpu.sync_copy(deg_sh.at[pl.ds(RS * NSUB, TAIL)],
                        degp_hbm.at[cid, pl.ds(RS * NSUB, TAIL)])


_sc_prep = pl.kernel(
    _sc_prep_body,
    out_type=(jax.ShapeDtypeStruct((NSC, N, LANES), jnp.float32),
              jax.ShapeDtypeStruct((N, NAME_D), jnp.float32)),
    mesh=_mesh,
    scratch_types=[
        pltpu.VMEM((NAMC,), jnp.int32),
        pltpu.VMEM((NAMC, NAME_D), jnp.float32),
        pltpu.VMEM((NBUF, EC), jnp.int32),
        pltpu.VMEM((NBUF, EC), jnp.float32),
        pltpu.VMEM((NBUF, EC, LANES), jnp.float32),
        pltpu.VMEM_SHARED((N, LANES), jnp.float32),
    ] + [pltpu.SemaphoreType.DMA] * 9,
    compiler_params=_sc_params,
)


def _sc_agg_body(ei_hbm, ew_hbm, h_hbm, z_hbm,
                 acc_hbm,
                 ep_v, ew_v, rows_v, acc_sh,
                 g0, g1, g2, g3, s0, s1, s2, s3,
                 e0, e1, e2, e3, e4, e5, e6, e7):
    cid = lax.axis_index("c")
    sid = lax.axis_index("s")
    gsem = (g0, g1, g2, g3)
    ssem = (s0, s1, s2, s3)
    esem = (e0, e1, e2, e3, e4, e5, e6, e7)

    r0 = sid * RS
    pltpu.sync_copy(z_hbm.at[pl.ds(r0, RS)], acc_sh.at[pl.ds(r0, RS)])

    @pl.when(sid == 0)
    def _():
        pltpu.sync_copy(z_hbm.at[pl.ds(RS * NSUB, TAIL)],
                        acc_sh.at[pl.ds(RS * NSUB, TAIL)])

    plsc.subcore_barrier()

    j0 = cid * NCHUNK_SC + sid

    def fetch_ep(t, c):
        base = (j0 + t * NSUB) * EC
        pltpu.async_copy(ei_hbm.at[:, pl.ds(base, EC)], ep_v.at[c], esem[c])
        pltpu.async_copy(ew_hbm.at[pl.ds(base, EC)], ew_v.at[c], esem[c])

    def start_gather(b, c):
        pltpu.make_async_copy(ei_hbm.at[:, pl.ds(0, EC)], ep_v.at[c],
                              esem[c]).wait()
        pltpu.make_async_copy(ew_hbm.at[pl.ds(0, EC)], ew_v.at[c],
                              esem[c]).wait()
        pltpu.async_copy(h_hbm.at[ep_v.at[c, 0]], rows_v.at[b], gsem[b])

    def drain_scatter(b, c):
        pltpu.make_async_copy(rows_v.at[b], acc_sh.at[ep_v.at[c, 1]],
                              ssem[b]).wait()

    def process(t, b, c, drain, fetch_next, gather_next):
        pltpu.make_async_copy(h_hbm.at[ep_v.at[c, 0]], rows_v.at[b],
                              gsem[b]).wait()

        @pl.loop(0, EC // LANES)
        def _(g):
            w_grp = ew_v[c, pl.ds(g * LANES, LANES)]
            e0_ = g * LANES
            for j in range(LANES):
                w16 = _bcast_lane(w_grp, j)
                e = e0_ + j
                for k in range(HID // LANES):
                    rows_v[b, e, pl.ds(k * LANES, LANES)] = (
                        rows_v[b, e, pl.ds(k * LANES, LANES)] * w16)

        pltpu.async_copy(rows_v.at[b], acc_sh.at[ep_v.at[c, 1]], ssem[b],
                         add=True)
        b2 = (b + 2) % NBUF
        if fetch_next:
            fetch_ep(t + 4, (c + 4) % NEP)
        if drain:
            drain_scatter(b2, (c - 2) % NEP)
        if gather_next:
            start_gather(b2, (c + 2) % NEP)

    for t in range(4):
        fetch_ep(t, t)
    start_gather(0, 0)
    start_gather(1, 1)
    for t in range(NEP):
        process(t, t % NBUF, t, t >= 2, True, True)

    @pl.loop(1, N_GROUPS)
    def _(q):
        t0 = q * NEP
        for p in range(NEP):
            process(t0 + p, p % NBUF, p, True, True, True)

    for t in range(TAIL_START, T_SUB):
        process(t, t % NBUF, t % NEP,
                t + 2 < T_SUB, t + 4 < T_SUB, t + 2 < T_SUB)

    for t in range(T_SUB - 4, T_SUB):
        drain_scatter(t % NBUF, t % NEP)

    plsc.subcore_barrier()
    pltpu.sync_copy(acc_sh.at[pl.ds(r0, RS)],
                    acc_hbm.at[cid, pl.ds(r0, RS)])

    @pl.when(sid == 0)
    def _():
        pltpu.sync_copy(acc_sh.at[pl.ds(RS * NSUB, TAIL)],
                        acc_hbm.at[cid, pl.ds(RS * NSUB, TAIL)])


_sc_agg = pl.kernel(
    _sc_agg_body,
    out_type=jax.ShapeDtypeStruct((NSC, N, HID), jnp.float32),
    mesh=_mesh,
    scratch_types=[
        pltpu.VMEM((NEP, 2, EC), jnp.int32),
        pltpu.VMEM((NEP, EC), jnp.float32),
        pltpu.VMEM((NBUF, EC, HID), jnp.float32),
        pltpu.VMEM_SHARED((N, HID), jnp.float32),
    ] + [pltpu.SemaphoreType.DMA] * 16,
    compiler_params=_sc_params,
)


def _tc1_body(degp_ref, nf_ref, xt_ref, xb_ref, tt_ref, w1_ref,
              dinv_ref, h2_ref):
    deg = degp_ref[0, :, 0:1] + degp_ref[1, :, 0:1] + 1.0
    dinv = lax.rsqrt(deg)
    dinv_ref[...] = dinv
    w1 = w1_ref[...]
    type_proj = jnp.dot(tt_ref[...], w1[NAME_D:NAME_D + TYPE_D, :],
                        preferred_element_type=jnp.float32)
    oh = jnp.where(
        lax.broadcasted_iota(jnp.int32, (N, TYPE_V), 1) == xt_ref[...],
        1.0, 0.0)
    xw = (jnp.dot(nf_ref[...], w1[:NAME_D, :],
                  preferred_element_type=jnp.float32)
          + jnp.dot(oh, type_proj, preferred_element_type=jnp.float32)
          + jnp.dot(xb_ref[...], w1[NAME_D + TYPE_D:, :],
                    preferred_element_type=jnp.float32))
    h2_ref[...] = dinv * xw


def _tc2_body(acc_ref, h2_ref, dinv_ref, b_ref, w2_ref, out_ref):
    dinv = dinv_ref[...]
    a = jnp.maximum(
        dinv * (acc_ref[0] + acc_ref[1] + h2_ref[...]) + b_ref[...], 0.0)
    out_ref[...] = dinv * jnp.dot(a, w2_ref[...],
                                  preferred_element_type=jnp.float32)


def _tc3_body(acc_ref, h2_ref, dinv_ref, b_ref, batch_ref, wc_ref, bc_ref,
              out_ref):
    a = jnp.maximum(
        dinv_ref[...] * (acc_ref[0] + acc_ref[1] + h2_ref[...]) + b_ref[...],
        0.0)
    oh = jnp.where(
        lax.broadcasted_iota(jnp.int32, (NGRAPH, N), 0) == batch_ref[...],
        1.0, 0.0)
    sums = jnp.dot(oh, a, preferred_element_type=jnp.float32)
    cnts = jnp.sum(oh, axis=1, keepdims=True)
    pooled = sums / jnp.maximum(cnts, 1.0)
    out_ref[...] = (jnp.dot(pooled, wc_ref[...],
                            preferred_element_type=jnp.float32) + bc_ref[...])


def kernel(x_names, x_types, x_behaviors, edge_index, edge_weight, batch,
           name_table, type_table, W1, b1, W2, b2, Wc, bc):
    ei = edge_index.astype(jnp.int32)
    ew = edge_weight.astype(jnp.float32)
    names = x_names.astype(jnp.int32)
    xt = x_types.astype(jnp.int32).reshape(N, 1)
    batch2 = batch.astype(jnp.int32).reshape(1, N)
    z16 = jnp.zeros((N, LANES), jnp.float32)
    z128 = jnp.zeros((N, HID), jnp.float32)
    ncls = Wc.shape[1]

    degp, nfeat = _sc_prep(ei, ew, names, name_table.astype(jnp.float32),
                           z16)

    dinv, h2 = pl.pallas_call(
        _tc1_body,
        out_shape=(jax.ShapeDtypeStruct((N, 1), jnp.float32),
                   jax.ShapeDtypeStruct((N, HID), jnp.float32)),
    )(degp, nfeat, xt, x_behaviors.astype(jnp.float32),
      type_table.astype(jnp.float32), W1)

    acc1 = _sc_agg(ei, ew, h2, z128)

    h2b = pl.pallas_call(
        _tc2_body,
        out_shape=jax.ShapeDtypeStruct((N, HID), jnp.float32),
    )(acc1, h2, dinv, b1.reshape(1, HID), W2)

    acc2 = _sc_agg(ei, ew, h2b, z128)

    out = pl.pallas_call(
        _tc3_body,
        out_shape=jax.ShapeDtypeStruct((NGRAPH, ncls), jnp.float32),
    )(acc2, h2b, dinv, b2.reshape(1, HID), batch2, Wc, bc.reshape(1, ncls))
    return out
